# Initial kernel scaffold; baseline (speedup 1.0000x reference)
import jax
import jax.numpy as jnp
from jax.experimental import pallas as pl


def kernel(x, edge_index, edge_pairs, W_in0, b_in0, g0, be0, W_in1, b_in1, W_gcn, b_gcn, g1, be1, W_l, b_l, W_r, g2, be2, W_jk, b_jk, W_d0, b_d0, W_d1, b_d1, W_d2, b_d2):
    raise NotImplementedError("write your pallas kernel here")


# speedup vs baseline: 5.8315x; 5.8315x over previous
import functools

import jax
import jax.numpy as jnp
from jax import lax
from jax.experimental import pallas as pl
from jax.experimental.pallas import tpu as pltpu
from jax.experimental.pallas import tpu_sc as plsc

_N = 10000
_E = 320000
_P = 100000
_NP = 10240
_EP = 327680
_ECH_T = 160
_PP = 100352
_PCH_T = 49

_mesh = plsc.VectorSubcoreMesh(core_axis_name="c", subcore_axis_name="s")
_f32 = jnp.float32


@functools.partial(
    pl.kernel,
    mesh=_mesh,
    out_type=jax.ShapeDtypeStruct((2 * _NP, 128), _f32),
    scratch_types=[
        pltpu.VMEM((_ECH_T, 128), jnp.int32),
        pltpu.VMEM((128, 128), _f32),
        pltpu.VMEM_SHARED((_NP, 128), _f32),
    ],
)
def _sc_hist(edge4d, hist_out, idx_v, buf, hist_spm):
    c = lax.axis_index("c")
    s = lax.axis_index("s")

    def _fill(val):
        def _fi(i, carry):
            for k in range(8):
                buf[i, pl.ds(k * 16, 16)] = jnp.full((16,), val, _f32)
            return carry

        lax.fori_loop(0, 128, _fi, 0)

    _fill(0.0)
    for k in range(5):
        pltpu.sync_copy(buf, hist_spm.at[pl.ds(s * 640 + k * 128, 128)])
    _fill(1.0)
    pltpu.sync_copy(edge4d.at[c, s], idx_v)
    plsc.subcore_barrier()

    def _body(j, carry):
        pltpu.sync_copy(buf, hist_spm.at[idx_v.at[j]], add=True)
        return carry

    lax.fori_loop(0, _ECH_T, _body, 0)
    plsc.subcore_barrier()
    pltpu.sync_copy(hist_spm.at[pl.ds(s * 640, 640)],
                    hist_out.at[pl.ds(c * _NP + s * 640, 640)])


@functools.partial(
    pl.kernel,
    mesh=_mesh,
    out_type=jax.ShapeDtypeStruct((2 * _NP, 128), _f32),
    scratch_types=[
        pltpu.VMEM((_ECH_T // 2, 128), jnp.int32),
        pltpu.VMEM((_ECH_T // 2, 128), jnp.int32),
        pltpu.VMEM((128, 128), _f32),
        pltpu.VMEM_SHARED((_NP, 128), _f32),
        pltpu.SemaphoreType.DMA,
    ],
)
def _sc_scatter(tab2n, edge4d, out2n, sidx, didx, rows, acc, sem):
    c = lax.axis_index("c")
    s = lax.axis_index("s")
    off = c * _NP

    def _zrow(i, carry):
        for k in range(8):
            rows[i, pl.ds(k * 16, 16)] = jnp.zeros((16,), _f32)
        return carry

    lax.fori_loop(0, 128, _zrow, 0)
    for k in range(5):
        pltpu.sync_copy(rows, acc.at[pl.ds(s * 640 + k * 128, 128)])
    plsc.subcore_barrier()

    def _body(j, carry):
        pltpu.async_copy(tab2n.at[sidx.at[j]], rows, sem).wait()
        pltpu.sync_copy(rows, acc.at[didx.at[j]], add=True)
        return carry

    def _adj(i, carry):
        for k in range(8):
            sidx[i, pl.ds(k * 16, 16)] = sidx[i, pl.ds(k * 16, 16)] + off
        return carry

    for h in range(2):
        pltpu.sync_copy(edge4d.at[0, s, pl.ds(h * 80, 80)], sidx)
        pltpu.sync_copy(edge4d.at[1, s, pl.ds(h * 80, 80)], didx)
        lax.fori_loop(0, 80, _adj, 0)
        lax.fori_loop(0, 80, _body, 0)
    plsc.subcore_barrier()
    pltpu.sync_copy(acc.at[pl.ds(s * 640, 640)],
                    out2n.at[pl.ds(c * _NP + s * 640, 640)])


@functools.partial(
    pl.kernel,
    mesh=_mesh,
    out_type=[
        jax.ShapeDtypeStruct((2, _PP, 128), _f32),
        jax.ShapeDtypeStruct((2, _PP, 128), _f32),
    ],
    scratch_types=[
        pltpu.VMEM((_PCH_T, 128), jnp.int32),
        pltpu.VMEM((_PCH_T, 128), jnp.int32),
        pltpu.VMEM((128, 128), _f32),
        pltpu.SemaphoreType.DMA,
    ],
)
def _sc_pair_gather(z2n, pair4d, out_a, out_b, aidx, bidx, rows, sem):
    c = lax.axis_index("c")
    s = lax.axis_index("s")
    off = c * _NP
    pltpu.sync_copy(pair4d.at[0, s], aidx)
    pltpu.sync_copy(pair4d.at[1, s], bidx)

    def _adj(i, carry):
        for k in range(8):
            aidx[i, pl.ds(k * 16, 16)] = aidx[i, pl.ds(k * 16, 16)] + off
            bidx[i, pl.ds(k * 16, 16)] = bidx[i, pl.ds(k * 16, 16)] + off
        return carry

    lax.fori_loop(0, _PCH_T, _adj, 0)

    def _body(j, carry):
        base = (s * _PCH_T + j) * 128
        pltpu.async_copy(z2n.at[aidx.at[j]], rows, sem).wait()
        pltpu.sync_copy(rows, out_a.at[c, pl.ds(base, 128)])
        pltpu.async_copy(z2n.at[bidx.at[j]], rows, sem).wait()
        pltpu.sync_copy(rows, out_b.at[c, pl.ds(base, 128)])
        return carry

    lax.fori_loop(0, _PCH_T, _body, 0)


def _ln_tc(h, g, b):
    mu = jnp.sum(h, axis=1, keepdims=True) * (1.0 / 256.0)
    d = h - mu
    var = jnp.sum(d * d, axis=1, keepdims=True) * (1.0 / 256.0)
    return d * lax.rsqrt(var + 1e-5) * g + b


def _median_body(deg_ref, med_ref):
    deg = deg_ref[:]
    flat = (lax.broadcasted_iota(jnp.int32, (80, 128), 0) * 128
            + lax.broadcasted_iota(jnp.int32, (80, 128), 1))
    valid = flat < _N

    def _body(i, lohi):
        lo, hi = lohi
        mid = (lo + hi) // 2
        cnt = jnp.sum(jnp.where(valid & (deg <= mid.astype(_f32)), 1, 0))
        ge = cnt >= (_N - 1) // 2 + 1
        return (jnp.where(ge, lo, mid + 1), jnp.where(ge, mid, hi))

    lo, _hi = lax.fori_loop(0, 19, _body,
                            (jnp.int32(0), jnp.int32(_E)))
    med_ref[0, 0] = lo.astype(_f32)


def _prep_body(x_ref, deg_ref, cnt_ref, med_ref, w0x_ref, w0f_ref, b0_ref,
               g0_ref, be0_ref, w1_ref, b1_ref, wg_ref,
               h0_ref, ulo_ref, uhi_ref, dinv_ref):
    xb = x_ref[:]
    degb = deg_ref[:]
    cntb = cnt_ref[:]
    med = med_ref[0, 0]
    nrm = jnp.sqrt(jnp.sum(xb * xb, axis=1, keepdims=True))
    xn = xb / jnp.maximum(nrm, 1e-12)
    f0 = degb / (jnp.float32(_E / _N) + jnp.float32(1e-6))
    f1 = jnp.log(degb + 1.0)
    f2 = lax.rsqrt(jnp.maximum(degb, 1.0))
    f3 = (degb > med).astype(_f32)
    t = jnp.dot(xn, w0x_ref[:], preferred_element_type=_f32, precision=lax.Precision.HIGHEST)
    t = (t + f0 * w0f_ref[0:1, :] + f1 * w0f_ref[1:2, :]
         + f2 * w0f_ref[2:3, :] + f3 * w0f_ref[3:4, :] + b0_ref[:])
    t = jnp.maximum(_ln_tc(t, g0_ref[:], be0_ref[:]), 0.0)
    h0 = jnp.dot(t, w1_ref[:], preferred_element_type=_f32, precision=lax.Precision.HIGHEST) + b1_ref[:]
    hw = jnp.dot(h0, wg_ref[:], preferred_element_type=_f32, precision=lax.Precision.HIGHEST)
    dinv = lax.rsqrt(cntb + 1.0)
    u = hw * dinv
    h0_ref[:] = h0
    ulo_ref[:] = u[:, :128]
    uhi_ref[:] = u[:, 128:]
    dinv_ref[:] = dinv


def _mid_body(ylo_ref, yhi_ref, ulo_ref, uhi_ref, h0_ref, dinv_ref, bg_ref,
              g1_ref, be1_ref, h1lo_ref, h1hi_ref):
    dinv = dinv_ref[:]
    agg = jnp.concatenate(
        [dinv * (ylo_ref[:] + ulo_ref[:]),
         dinv * (yhi_ref[:] + uhi_ref[:])], axis=1) + bg_ref[:]
    h1 = h0_ref[:] + jnp.maximum(_ln_tc(agg, g1_ref[:], be1_ref[:]), 0.0)
    h1lo_ref[:] = h1[:, :128]
    h1hi_ref[:] = h1[:, 128:]


def _post_body(slo_ref, shi_ref, cnt_ref, h0_ref, h1lo_ref, h1hi_ref,
               wl_ref, bl_ref, wr_ref, g2_ref, be2_ref,
               wjk0_ref, wjk1_ref, wjk2_ref, bjk_ref, zlo_ref, zhi_ref):
    inv_cnt = 1.0 / jnp.maximum(cnt_ref[:], 1.0)
    mean = jnp.concatenate([slo_ref[:], shi_ref[:]], axis=1) * inv_cnt
    h1 = jnp.concatenate([h1lo_ref[:], h1hi_ref[:]], axis=1)
    sage = (jnp.dot(mean, wl_ref[:], preferred_element_type=_f32, precision=lax.Precision.HIGHEST) + bl_ref[:]
            + jnp.dot(h1, wr_ref[:], preferred_element_type=_f32, precision=lax.Precision.HIGHEST))
    h2 = h1 + jnp.maximum(_ln_tc(sage, g2_ref[:], be2_ref[:]), 0.0)
    z = (jnp.dot(h0_ref[:], wjk0_ref[:], preferred_element_type=_f32, precision=lax.Precision.HIGHEST)
         + jnp.dot(h1, wjk1_ref[:], preferred_element_type=_f32, precision=lax.Precision.HIGHEST)
         + jnp.dot(h2, wjk2_ref[:], preferred_element_type=_f32, precision=lax.Precision.HIGHEST)
         + bjk_ref[:])
    zlo_ref[:] = z[:, :128]
    zhi_ref[:] = z[:, 128:]


def _dec_body(alo_ref, ahi_ref, blo_ref, bhi_ref, wd0l_ref, wd0h_ref,
              bd0_ref, wd1_ref, bd1_ref, wd2_ref, bd2_ref, out_ref):
    hlo = alo_ref[:] * blo_ref[:]
    hhi = ahi_ref[:] * bhi_ref[:]
    t = (jnp.dot(hlo, wd0l_ref[:], preferred_element_type=_f32, precision=lax.Precision.HIGHEST)
         + jnp.dot(hhi, wd0h_ref[:], preferred_element_type=_f32, precision=lax.Precision.HIGHEST)
         + bd0_ref[:])
    t = jnp.maximum(t, 0.0)
    t = jnp.dot(t, wd1_ref[:], preferred_element_type=_f32, precision=lax.Precision.HIGHEST) + bd1_ref[:]
    t = jnp.maximum(t, 0.0)
    out_ref[:] = (jnp.sum(t * wd2_ref[:], axis=1, keepdims=True)
                  + bd2_ref[0, 0])


def _full(shape):
    return pl.BlockSpec(shape, lambda *i: tuple(0 for _ in shape))


def _rows(shape):
    return pl.BlockSpec(shape, lambda i: (i,) + tuple(0 for _ in shape[1:]))


def kernel(x, edge_index, edge_pairs, W_in0, b_in0, g0, be0, W_in1, b_in1,
           W_gcn, b_gcn, g1, be1, W_l, b_l, W_r, g2, be2, W_jk, b_jk,
           W_d0, b_d0, W_d1, b_d1, W_d2, b_d2):
    pad_e = 10000 + (jnp.arange(_EP - _E, dtype=jnp.int32) % 240)
    edge4d = jnp.concatenate(
        [edge_index, jnp.stack([pad_e, pad_e])], axis=1
    ).reshape(2, 16, _ECH_T, 128)
    pad_p = 10000 + (jnp.arange(_PP - _P, dtype=jnp.int32) % 240)
    pair4d = jnp.stack(
        [jnp.concatenate([edge_pairs[:, 0], pad_p]),
         jnp.concatenate([edge_pairs[:, 1], pad_p])]
    ).reshape(2, 16, _PCH_T, 128)
    xp = jnp.pad(x, ((0, _NP - _N), (0, 0)))

    w0x = W_in0[:128]
    w0f = jnp.pad(W_in0[128:], ((0, 4), (0, 0)))
    row = lambda v: v.reshape(1, -1)

    hist = _sc_hist(edge4d)
    deg, cnt = hist[:_NP, 0], hist[_NP:, 0]
    deg1 = deg.reshape(_NP, 1)
    cnt1 = cnt.reshape(_NP, 1)

    med = pl.pallas_call(
        _median_body,
        out_shape=jax.ShapeDtypeStruct((1, 1), _f32),
        in_specs=[_full((80, 128))],
        out_specs=pl.BlockSpec(memory_space=pltpu.SMEM),
    )(deg.reshape(80, 128))

    grid = (_NP // 256,)
    h0, u_lo, u_hi, dinv = pl.pallas_call(
        _prep_body,
        grid=grid,
        out_shape=[
            jax.ShapeDtypeStruct((_NP, 256), _f32),
            jax.ShapeDtypeStruct((_NP, 128), _f32),
            jax.ShapeDtypeStruct((_NP, 128), _f32),
            jax.ShapeDtypeStruct((_NP, 1), _f32),
        ],
        in_specs=[
            _rows((256, 128)), _rows((256, 1)), _rows((256, 1)),
            pl.BlockSpec(memory_space=pltpu.SMEM),
            _full((128, 256)), _full((8, 256)),
            _full((1, 256)), _full((1, 256)), _full((1, 256)),
            _full((256, 256)), _full((1, 256)), _full((256, 256)),
        ],
        out_specs=[
            _rows((256, 256)), _rows((256, 128)), _rows((256, 128)),
            _rows((256, 1)),
        ],
    )(xp, deg1, cnt1, med, w0x, w0f, row(b_in0), row(g0), row(be0),
      W_in1, row(b_in1), W_gcn)

    y2n = _sc_scatter(jnp.concatenate([u_lo, u_hi], axis=0), edge4d)
    y_lo, y_hi = y2n[:_NP], y2n[_NP:]

    h1_lo, h1_hi = pl.pallas_call(
        _mid_body,
        grid=grid,
        out_shape=[
            jax.ShapeDtypeStruct((_NP, 128), _f32),
            jax.ShapeDtypeStruct((_NP, 128), _f32),
        ],
        in_specs=[
            _rows((256, 128)), _rows((256, 128)), _rows((256, 128)),
            _rows((256, 128)), _rows((256, 256)), _rows((256, 1)),
            _full((1, 256)), _full((1, 256)), _full((1, 256)),
        ],
        out_specs=[_rows((256, 128)), _rows((256, 128))],
    )(y_lo, y_hi, u_lo, u_hi, h0, dinv, row(b_gcn), row(g1), row(be1))

    s2n = _sc_scatter(jnp.concatenate([h1_lo, h1_hi], axis=0), edge4d)
    s_lo, s_hi = s2n[:_NP], s2n[_NP:]

    z_lo, z_hi = pl.pallas_call(
        _post_body,
        grid=grid,
        out_shape=[
            jax.ShapeDtypeStruct((_NP, 128), _f32),
            jax.ShapeDtypeStruct((_NP, 128), _f32),
        ],
        in_specs=[
            _rows((256, 128)), _rows((256, 128)), _rows((256, 1)),
            _rows((256, 256)), _rows((256, 128)), _rows((256, 128)),
            _full((256, 256)), _full((1, 256)), _full((256, 256)),
            _full((1, 256)), _full((1, 256)),
            _full((256, 256)), _full((256, 256)), _full((256, 256)),
            _full((1, 256)),
        ],
        out_specs=[_rows((256, 128)), _rows((256, 128))],
    )(s_lo, s_hi, cnt1, h0, h1_lo, h1_hi, W_l, row(b_l), W_r, row(g2),
      row(be2), W_jk[:256], W_jk[256:512], W_jk[512:], row(b_jk))

    ga, gb = _sc_pair_gather(jnp.concatenate([z_lo, z_hi], axis=0), pair4d)
    za_lo, za_hi, zb_lo, zb_hi = ga[0], ga[1], gb[0], gb[1]

    out = pl.pallas_call(
        _dec_body,
        grid=(_PP // 512,),
        out_shape=jax.ShapeDtypeStruct((_PP, 1), _f32),
        in_specs=[
            _rows((512, 128)), _rows((512, 128)), _rows((512, 128)),
            _rows((512, 128)),
            _full((128, 256)), _full((128, 256)), _full((1, 256)),
            _full((256, 128)), _full((1, 128)), _full((1, 128)),
            _full((1, 1)),
        ],
        out_specs=_rows((512, 1)),
    )(za_lo, za_hi, zb_lo, zb_hi, W_d0[:128], W_d0[128:], row(b_d0),
      W_d1, row(b_d1), W_d2.reshape(1, 128), b_d2.reshape(1, 1))

    return out[:_P, 0]


# speedup vs baseline: 7.3214x; 1.2555x over previous
import functools

import jax
import jax.numpy as jnp
from jax import lax
from jax.experimental import pallas as pl
from jax.experimental.pallas import tpu as pltpu
from jax.experimental.pallas import tpu_sc as plsc

_N = 10000
_E = 320000
_P = 100000
_NP = 10240
_EP = 327680
_ECH_T = 160
_PP = 100352
_PCH_T = 49

_mesh = plsc.VectorSubcoreMesh(core_axis_name="c", subcore_axis_name="s")
_f32 = jnp.float32


@functools.partial(
    pl.kernel,
    mesh=_mesh,
    out_type=jax.ShapeDtypeStruct((2 * _NP, 128), _f32),
    scratch_types=[
        pltpu.VMEM((_ECH_T, 128), jnp.int32),
        pltpu.VMEM((128, 128), _f32),
        pltpu.VMEM_SHARED((_NP, 128), _f32),
    ],
)
def _sc_hist(edge4d, hist_out, idx_v, buf, hist_spm):
    c = lax.axis_index("c")
    s = lax.axis_index("s")

    def _fill(val):
        def _fi(i, carry):
            for k in range(8):
                buf[i, pl.ds(k * 16, 16)] = jnp.full((16,), val, _f32)
            return carry

        lax.fori_loop(0, 128, _fi, 0)

    _fill(0.0)
    for k in range(5):
        pltpu.sync_copy(buf, hist_spm.at[pl.ds(s * 640 + k * 128, 128)])
    _fill(1.0)
    pltpu.sync_copy(edge4d.at[c, s], idx_v)
    plsc.subcore_barrier()

    def _body(j, carry):
        pltpu.sync_copy(buf, hist_spm.at[idx_v.at[j]], add=True)
        return carry

    lax.fori_loop(0, _ECH_T, _body, 0)
    plsc.subcore_barrier()
    pltpu.sync_copy(hist_spm.at[pl.ds(s * 640, 640)],
                    hist_out.at[pl.ds(c * _NP + s * 640, 640)])


@functools.partial(
    pl.kernel,
    mesh=_mesh,
    out_type=jax.ShapeDtypeStruct((2 * _NP, 128), _f32),
    scratch_types=[
        pltpu.VMEM((40, 128), jnp.int32),
        pltpu.VMEM((40, 128), jnp.int32),
        pltpu.VMEM((128, 128), _f32),
        pltpu.VMEM((128, 128), _f32),
        pltpu.VMEM_SHARED((_NP, 128), _f32),
        pltpu.SemaphoreType.DMA,
        pltpu.SemaphoreType.DMA,
    ],
)
def _sc_scatter(tab2n, edge4d, out2n, sidx, didx, rows0, rows1, acc,
                semA, semB):
    c = lax.axis_index("c")
    s = lax.axis_index("s")
    off = c * _NP

    def _zrow(i, carry):
        for k in range(8):
            rows0[i, pl.ds(k * 16, 16)] = jnp.zeros((16,), _f32)
        return carry

    lax.fori_loop(0, 128, _zrow, 0)
    for k in range(5):
        pltpu.sync_copy(rows0, acc.at[pl.ds(s * 640 + k * 128, 128)])
    plsc.subcore_barrier()

    def _adj(i, carry):
        for k in range(8):
            sidx[i, pl.ds(k * 16, 16)] = sidx[i, pl.ds(k * 16, 16)] + off
        return carry

    def _body(i, carry):
        j = 2 * i
        pltpu.make_async_copy(tab2n.at[sidx.at[j]], rows0, semA).wait()
        pltpu.async_copy(tab2n.at[sidx.at[j + 1]], rows1, semB)
        pltpu.sync_copy(rows0, acc.at[didx.at[j]], add=True)
        pltpu.make_async_copy(tab2n.at[sidx.at[j + 1]], rows1, semB).wait()

        @pl.when(j + 2 < 40)
        def _():
            pltpu.async_copy(tab2n.at[sidx.at[j + 2]], rows0, semA)

        pltpu.sync_copy(rows1, acc.at[didx.at[j + 1]], add=True)
        return carry

    for q in range(4):
        pltpu.sync_copy(edge4d.at[0, s, pl.ds(q * 40, 40)], sidx)
        pltpu.sync_copy(edge4d.at[1, s, pl.ds(q * 40, 40)], didx)
        lax.fori_loop(0, 40, _adj, 0)
        pltpu.async_copy(tab2n.at[sidx.at[0]], rows0, semA)
        lax.fori_loop(0, 20, _body, 0)

    plsc.subcore_barrier()
    pltpu.sync_copy(acc.at[pl.ds(s * 640, 640)],
                    out2n.at[pl.ds(c * _NP + s * 640, 640)])


@functools.partial(
    pl.kernel,
    mesh=_mesh,
    out_type=[
        jax.ShapeDtypeStruct((2, _PP, 128), _f32),
        jax.ShapeDtypeStruct((2, _PP, 128), _f32),
    ],
    scratch_types=[
        pltpu.VMEM((_PCH_T, 128), jnp.int32),
        pltpu.VMEM((_PCH_T, 128), jnp.int32),
        pltpu.VMEM((128, 128), _f32),
        pltpu.VMEM((128, 128), _f32),
        pltpu.SemaphoreType.DMA,
        pltpu.SemaphoreType.DMA,
    ],
)
def _sc_pair_gather(z2n, pair4d, out_a, out_b, aidx, bidx, rows0, rows1,
                    semA, semB):
    c = lax.axis_index("c")
    s = lax.axis_index("s")
    off = c * _NP
    pltpu.sync_copy(pair4d.at[0, s], aidx)
    pltpu.sync_copy(pair4d.at[1, s], bidx)

    def _adj(i, carry):
        for k in range(8):
            aidx[i, pl.ds(k * 16, 16)] = aidx[i, pl.ds(k * 16, 16)] + off
            bidx[i, pl.ds(k * 16, 16)] = bidx[i, pl.ds(k * 16, 16)] + off
        return carry

    lax.fori_loop(0, _PCH_T, _adj, 0)
    pltpu.async_copy(z2n.at[aidx.at[0]], rows0, semA)

    def _body(j, carry):
        base = (s * _PCH_T + j) * 128
        pltpu.make_async_copy(z2n.at[aidx.at[j]], rows0, semA).wait()
        pltpu.async_copy(z2n.at[bidx.at[j]], rows1, semB)
        pltpu.sync_copy(rows0, out_a.at[c, pl.ds(base, 128)])
        pltpu.make_async_copy(z2n.at[bidx.at[j]], rows1, semB).wait()

        @pl.when(j + 1 < _PCH_T)
        def _():
            pltpu.async_copy(z2n.at[aidx.at[j + 1]], rows0, semA)

        pltpu.sync_copy(rows1, out_b.at[c, pl.ds(base, 128)])
        return carry

    lax.fori_loop(0, _PCH_T, _body, 0)


_HI = lax.Precision.HIGHEST


def _ln_tc(h, g, b):
    mu = jnp.sum(h, axis=1, keepdims=True) * (1.0 / 256.0)
    d = h - mu
    var = jnp.sum(d * d, axis=1, keepdims=True) * (1.0 / 256.0)
    return d * lax.rsqrt(var + 1e-5) * g + b


def _median_body(deg_ref, med_ref):
    deg = deg_ref[:]
    flat = (lax.broadcasted_iota(jnp.int32, (80, 128), 0) * 128
            + lax.broadcasted_iota(jnp.int32, (80, 128), 1))
    valid = flat < _N

    def _body(i, lohi):
        lo, hi = lohi
        mid = (lo + hi) // 2
        cnt = jnp.sum(jnp.where(valid & (deg <= mid.astype(_f32)), 1, 0))
        ge = cnt >= (_N - 1) // 2 + 1
        return (jnp.where(ge, lo, mid + 1), jnp.where(ge, mid, hi))

    lo, _hi = lax.fori_loop(0, 19, _body,
                            (jnp.int32(0), jnp.int32(_E)))
    med_ref[0, 0] = lo.astype(_f32)


def _prep_body(x_ref, deg_ref, cnt_ref, med_ref, w0x_ref, w0f_ref, b0_ref,
               g0_ref, be0_ref, w1_ref, b1_ref, wg_ref,
               h0_ref, u2_ref, dinv_ref):
    xb = x_ref[:]
    degb = deg_ref[:]
    cntb = cnt_ref[:]
    med = med_ref[0, 0]
    nrm = jnp.sqrt(jnp.sum(xb * xb, axis=1, keepdims=True))
    xn = xb / jnp.maximum(nrm, 1e-12)
    f0 = degb / (jnp.float32(_E / _N) + jnp.float32(1e-6))
    f1 = jnp.log(degb + 1.0)
    f2 = lax.rsqrt(jnp.maximum(degb, 1.0))
    f3 = (degb > med).astype(_f32)
    t = jnp.dot(xn, w0x_ref[:], preferred_element_type=_f32, precision=_HI)
    t = (t + f0 * w0f_ref[0:1, :] + f1 * w0f_ref[1:2, :]
         + f2 * w0f_ref[2:3, :] + f3 * w0f_ref[3:4, :] + b0_ref[:])
    t = jnp.maximum(_ln_tc(t, g0_ref[:], be0_ref[:]), 0.0)
    h0 = jnp.dot(t, w1_ref[:], preferred_element_type=_f32,
                 precision=_HI) + b1_ref[:]
    hw = jnp.dot(h0, wg_ref[:], preferred_element_type=_f32, precision=_HI)
    dinv = lax.rsqrt(cntb + 1.0)
    u = hw * dinv
    h0_ref[:] = h0
    u2_ref[0] = u[:, :128]
    u2_ref[1] = u[:, 128:]
    dinv_ref[:] = dinv


def _mid_body(y2_ref, u2_ref, h0_ref, dinv_ref, bg_ref,
              g1_ref, be1_ref, h12_ref):
    dinv = dinv_ref[:]
    agg = jnp.concatenate(
        [dinv * (y2_ref[0] + u2_ref[0]),
         dinv * (y2_ref[1] + u2_ref[1])], axis=1) + bg_ref[:]
    h1 = h0_ref[:] + jnp.maximum(_ln_tc(agg, g1_ref[:], be1_ref[:]), 0.0)
    h12_ref[0] = h1[:, :128]
    h12_ref[1] = h1[:, 128:]


def _post_body(s2_ref, cnt_ref, h0_ref, h12_ref,
               wl_ref, bl_ref, wr_ref, g2_ref, be2_ref,
               wjk0_ref, wjk1_ref, wjk2_ref, bjk_ref, z2_ref):
    inv_cnt = 1.0 / jnp.maximum(cnt_ref[:], 1.0)
    mean = jnp.concatenate([s2_ref[0], s2_ref[1]], axis=1) * inv_cnt
    h1 = jnp.concatenate([h12_ref[0], h12_ref[1]], axis=1)
    sage = (jnp.dot(mean, wl_ref[:], preferred_element_type=_f32,
                    precision=_HI) + bl_ref[:]
            + jnp.dot(h1, wr_ref[:], preferred_element_type=_f32,
                      precision=_HI))
    h2 = h1 + jnp.maximum(_ln_tc(sage, g2_ref[:], be2_ref[:]), 0.0)
    z = (jnp.dot(h0_ref[:], wjk0_ref[:], preferred_element_type=_f32,
                 precision=_HI)
         + jnp.dot(h1, wjk1_ref[:], preferred_element_type=_f32,
                   precision=_HI)
         + jnp.dot(h2, wjk2_ref[:], preferred_element_type=_f32,
                   precision=_HI)
         + bjk_ref[:])
    z2_ref[0] = z[:, :128]
    z2_ref[1] = z[:, 128:]


def _dec_body(ga_ref, gb_ref, wd0l_ref, wd0h_ref,
              bd0_ref, wd1_ref, bd1_ref, wd2_ref, bd2_ref, out_ref):
    hlo = ga_ref[0] * gb_ref[0]
    hhi = ga_ref[1] * gb_ref[1]
    t = (jnp.dot(hlo, wd0l_ref[:], preferred_element_type=_f32,
                 precision=_HI)
         + jnp.dot(hhi, wd0h_ref[:], preferred_element_type=_f32,
                   precision=_HI)
         + bd0_ref[:])
    t = jnp.maximum(t, 0.0)
    t = jnp.dot(t, wd1_ref[:], preferred_element_type=_f32,
                precision=_HI) + bd1_ref[:]
    t = jnp.maximum(t, 0.0)
    out_ref[:] = (jnp.sum(t * wd2_ref[:], axis=1, keepdims=True)
                  + bd2_ref[0, 0])


def _full(shape):
    return pl.BlockSpec(shape, lambda *i: tuple(0 for _ in shape))


def _rows(shape):
    return pl.BlockSpec(shape, lambda i: (i,) + tuple(0 for _ in shape[1:]))


def _stk(shape):
    return pl.BlockSpec(shape, lambda i: (0, i, 0))


def kernel(x, edge_index, edge_pairs, W_in0, b_in0, g0, be0, W_in1, b_in1,
           W_gcn, b_gcn, g1, be1, W_l, b_l, W_r, g2, be2, W_jk, b_jk,
           W_d0, b_d0, W_d1, b_d1, W_d2, b_d2):
    pad_e = 10000 + (jnp.arange(_EP - _E, dtype=jnp.int32) % 240)
    edge4d = jnp.concatenate(
        [edge_index, jnp.stack([pad_e, pad_e])], axis=1
    ).reshape(2, 16, _ECH_T, 128)
    pad_p = 10000 + (jnp.arange(_PP - _P, dtype=jnp.int32) % 240)
    pair4d = jnp.stack(
        [jnp.concatenate([edge_pairs[:, 0], pad_p]),
         jnp.concatenate([edge_pairs[:, 1], pad_p])]
    ).reshape(2, 16, _PCH_T, 128)
    xp = jnp.pad(x, ((0, _NP - _N), (0, 0)))

    w0x = W_in0[:128]
    w0f = jnp.pad(W_in0[128:], ((0, 4), (0, 0)))
    row = lambda v: v.reshape(1, -1)

    hist = _sc_hist(edge4d)
    deg, cnt = hist[:_NP, 0], hist[_NP:, 0]
    deg1 = deg.reshape(_NP, 1)
    cnt1 = cnt.reshape(_NP, 1)

    med = pl.pallas_call(
        _median_body,
        out_shape=jax.ShapeDtypeStruct((1, 1), _f32),
        in_specs=[_full((80, 128))],
        out_specs=pl.BlockSpec(memory_space=pltpu.SMEM),
    )(deg.reshape(80, 128))

    grid = (_NP // 256,)
    h0, u2, dinv = pl.pallas_call(
        _prep_body,
        grid=grid,
        out_shape=[
            jax.ShapeDtypeStruct((_NP, 256), _f32),
            jax.ShapeDtypeStruct((2, _NP, 128), _f32),
            jax.ShapeDtypeStruct((_NP, 1), _f32),
        ],
        in_specs=[
            _rows((256, 128)), _rows((256, 1)), _rows((256, 1)),
            pl.BlockSpec(memory_space=pltpu.SMEM),
            _full((128, 256)), _full((8, 256)),
            _full((1, 256)), _full((1, 256)), _full((1, 256)),
            _full((256, 256)), _full((1, 256)), _full((256, 256)),
        ],
        out_specs=[
            _rows((256, 256)), _stk((2, 256, 128)), _rows((256, 1)),
        ],
    )(xp, deg1, cnt1, med, w0x, w0f, row(b_in0), row(g0), row(be0),
      W_in1, row(b_in1), W_gcn)

    y2n = _sc_scatter(u2.reshape(2 * _NP, 128), edge4d)

    h12 = pl.pallas_call(
        _mid_body,
        grid=grid,
        out_shape=jax.ShapeDtypeStruct((2, _NP, 128), _f32),
        in_specs=[
            _stk((2, 256, 128)), _stk((2, 256, 128)),
            _rows((256, 256)), _rows((256, 1)),
            _full((1, 256)), _full((1, 256)), _full((1, 256)),
        ],
        out_specs=_stk((2, 256, 128)),
    )(y2n.reshape(2, _NP, 128), u2, h0, dinv, row(b_gcn), row(g1), row(be1))

    s2n = _sc_scatter(h12.reshape(2 * _NP, 128), edge4d)

    z2 = pl.pallas_call(
        _post_body,
        grid=grid,
        out_shape=jax.ShapeDtypeStruct((2, _NP, 128), _f32),
        in_specs=[
            _stk((2, 256, 128)), _rows((256, 1)),
            _rows((256, 256)), _stk((2, 256, 128)),
            _full((256, 256)), _full((1, 256)), _full((256, 256)),
            _full((1, 256)), _full((1, 256)),
            _full((256, 256)), _full((256, 256)), _full((256, 256)),
            _full((1, 256)),
        ],
        out_specs=_stk((2, 256, 128)),
    )(s2n.reshape(2, _NP, 128), cnt1, h0, h12, W_l, row(b_l), W_r, row(g2),
      row(be2), W_jk[:256], W_jk[256:512], W_jk[512:], row(b_jk))

    ga, gb = _sc_pair_gather(z2.reshape(2 * _NP, 128), pair4d)

    out = pl.pallas_call(
        _dec_body,
        grid=(_PP // 512,),
        out_shape=jax.ShapeDtypeStruct((_PP, 1), _f32),
        in_specs=[
            _stk((2, 512, 128)), _stk((2, 512, 128)),
            _full((128, 256)), _full((128, 256)), _full((1, 256)),
            _full((256, 128)), _full((1, 128)), _full((1, 128)),
            _full((1, 1)),
        ],
        out_specs=_rows((512, 1)),
    )(ga, gb, W_d0[:128], W_d0[128:], row(b_d0),
      W_d1, row(b_d1), W_d2.reshape(1, 128), b_d2.reshape(1, 1))

    return out[:_P, 0]


# speedup vs baseline: 8.8692x; 1.2114x over previous
import functools

import jax
import jax.numpy as jnp
from jax import lax
from jax.experimental import pallas as pl
from jax.experimental.pallas import tpu as pltpu
from jax.experimental.pallas import tpu_sc as plsc

_N = 10000
_E = 320000
_P = 100000
_NP = 10240
_EP = 327680
_ECH_T = 160
_PP = 100352
_PCH_T = 49

_mesh = plsc.VectorSubcoreMesh(core_axis_name="c", subcore_axis_name="s")
_f32 = jnp.float32


@functools.partial(
    pl.kernel,
    mesh=_mesh,
    out_type=jax.ShapeDtypeStruct((2 * _NP, 128), _f32),
    scratch_types=[
        pltpu.VMEM((_ECH_T, 128), jnp.int32),
        pltpu.VMEM((128, 128), _f32),
        pltpu.VMEM_SHARED((_NP, 128), _f32),
    ],
)
def _sc_hist(edge4d, hist_out, idx_v, buf, hist_spm):
    c = lax.axis_index("c")
    s = lax.axis_index("s")

    def _fill(val):
        def _fi(i, carry):
            for k in range(8):
                buf[i, pl.ds(k * 16, 16)] = jnp.full((16,), val, _f32)
            return carry

        lax.fori_loop(0, 128, _fi, 0)

    _fill(0.0)
    for k in range(5):
        pltpu.sync_copy(buf, hist_spm.at[pl.ds(s * 640 + k * 128, 128)])
    _fill(1.0)
    pltpu.sync_copy(edge4d.at[c, s], idx_v)
    plsc.subcore_barrier()

    def _body(j, carry):
        pltpu.sync_copy(buf, hist_spm.at[idx_v.at[j]], add=True)
        return carry

    lax.fori_loop(0, _ECH_T, _body, 0)
    plsc.subcore_barrier()
    pltpu.sync_copy(hist_spm.at[pl.ds(s * 640, 640)],
                    hist_out.at[pl.ds(c * _NP + s * 640, 640)])


@functools.partial(
    pl.kernel,
    mesh=_mesh,
    out_type=jax.ShapeDtypeStruct((2 * _NP, 128), _f32),
    scratch_types=[
        pltpu.VMEM((40, 128), jnp.int32),
        pltpu.VMEM((40, 128), jnp.int32),
        pltpu.VMEM((128, 128), _f32),
        pltpu.VMEM((128, 128), _f32),
        pltpu.VMEM_SHARED((_NP, 128), _f32),
        pltpu.SemaphoreType.DMA,
        pltpu.SemaphoreType.DMA,
    ],
)
def _sc_scatter(tab2n, edge4d, out2n, sidx, didx, rows0, rows1, acc,
                semA, semB):
    c = lax.axis_index("c")
    s = lax.axis_index("s")
    off = c * _NP

    def _zrow(i, carry):
        for k in range(8):
            rows0[i, pl.ds(k * 16, 16)] = jnp.zeros((16,), _f32)
        return carry

    lax.fori_loop(0, 128, _zrow, 0)
    for k in range(5):
        pltpu.sync_copy(rows0, acc.at[pl.ds(s * 640 + k * 128, 128)])
    plsc.subcore_barrier()

    def _adj(i, carry):
        for k in range(8):
            sidx[i, pl.ds(k * 16, 16)] = sidx[i, pl.ds(k * 16, 16)] + off
        return carry

    def _body(i, carry):
        j = 2 * i
        pltpu.make_async_copy(tab2n.at[sidx.at[j]], rows0, semA).wait()
        pltpu.async_copy(tab2n.at[sidx.at[j + 1]], rows1, semB)
        pltpu.sync_copy(rows0, acc.at[didx.at[j]], add=True)
        pltpu.make_async_copy(tab2n.at[sidx.at[j + 1]], rows1, semB).wait()

        @pl.when(j + 2 < 40)
        def _():
            pltpu.async_copy(tab2n.at[sidx.at[j + 2]], rows0, semA)

        pltpu.sync_copy(rows1, acc.at[didx.at[j + 1]], add=True)
        return carry

    for q in range(4):
        pltpu.sync_copy(edge4d.at[0, s, pl.ds(q * 40, 40)], sidx)
        pltpu.sync_copy(edge4d.at[1, s, pl.ds(q * 40, 40)], didx)
        lax.fori_loop(0, 40, _adj, 0)
        pltpu.async_copy(tab2n.at[sidx.at[0]], rows0, semA)
        lax.fori_loop(0, 20, _body, 0)

    plsc.subcore_barrier()
    pltpu.sync_copy(acc.at[pl.ds(s * 640, 640)],
                    out2n.at[pl.ds(c * _NP + s * 640, 640)])


@functools.partial(
    pl.kernel,
    mesh=_mesh,
    out_type=[
        jax.ShapeDtypeStruct((2, _PP, 128), _f32),
        jax.ShapeDtypeStruct((2, _PP, 128), _f32),
    ],
    scratch_types=[
        pltpu.VMEM((_PCH_T, 128), jnp.int32),
        pltpu.VMEM((_PCH_T, 128), jnp.int32),
        pltpu.VMEM((128, 128), _f32),
        pltpu.VMEM((128, 128), _f32),
        pltpu.SemaphoreType.DMA,
        pltpu.SemaphoreType.DMA,
    ],
)
def _sc_pair_gather(z2n, pair4d, out_a, out_b, aidx, bidx, rows0, rows1,
                    semA, semB):
    c = lax.axis_index("c")
    s = lax.axis_index("s")
    off = c * _NP
    pltpu.sync_copy(pair4d.at[0, s], aidx)
    pltpu.sync_copy(pair4d.at[1, s], bidx)

    def _adj(i, carry):
        for k in range(8):
            aidx[i, pl.ds(k * 16, 16)] = aidx[i, pl.ds(k * 16, 16)] + off
            bidx[i, pl.ds(k * 16, 16)] = bidx[i, pl.ds(k * 16, 16)] + off
        return carry

    lax.fori_loop(0, _PCH_T, _adj, 0)
    pltpu.async_copy(z2n.at[aidx.at[0]], rows0, semA)

    def _body(j, carry):
        base = (s * _PCH_T + j) * 128
        pltpu.make_async_copy(z2n.at[aidx.at[j]], rows0, semA).wait()
        pltpu.async_copy(z2n.at[bidx.at[j]], rows1, semB)
        pltpu.sync_copy(rows0, out_a.at[c, pl.ds(base, 128)])
        pltpu.make_async_copy(z2n.at[bidx.at[j]], rows1, semB).wait()

        @pl.when(j + 1 < _PCH_T)
        def _():
            pltpu.async_copy(z2n.at[aidx.at[j + 1]], rows0, semA)

        pltpu.sync_copy(rows1, out_b.at[c, pl.ds(base, 128)])
        return carry

    lax.fori_loop(0, _PCH_T, _body, 0)


_HI = lax.Precision.HIGHEST


def _ln_tc(h, g, b):
    mu = jnp.sum(h, axis=1, keepdims=True) * (1.0 / 256.0)
    d = h - mu
    var = jnp.sum(d * d, axis=1, keepdims=True) * (1.0 / 256.0)
    return d * lax.rsqrt(var + 1e-5) * g + b


def _median_body(deg_ref, med_ref):
    deg = deg_ref[:]
    flat = (lax.broadcasted_iota(jnp.int32, (80, 128), 0) * 128
            + lax.broadcasted_iota(jnp.int32, (80, 128), 1))
    valid = flat < _N

    def _body(i, lohi):
        lo, hi = lohi
        mid = (lo + hi) // 2
        cnt = jnp.sum(jnp.where(valid & (deg <= mid.astype(_f32)), 1, 0))
        ge = cnt >= (_N - 1) // 2 + 1
        return (jnp.where(ge, lo, mid + 1), jnp.where(ge, mid, hi))

    lo, _hi = lax.fori_loop(0, 19, _body,
                            (jnp.int32(0), jnp.int32(_E)))
    med_ref[0, 0] = lo.astype(_f32)


def _prep_body(x_ref, deg_ref, cnt_ref, med_ref, w0x_ref, w0f_ref, b0_ref,
               g0_ref, be0_ref, w1_ref, b1_ref, wg_ref,
               h0_ref, u2_ref, dinv_ref):
    xb = x_ref[:]
    degb = deg_ref[:]
    cntb = cnt_ref[:]
    med = med_ref[0, 0]
    nrm = jnp.sqrt(jnp.sum(xb * xb, axis=1, keepdims=True))
    xn = xb / jnp.maximum(nrm, 1e-12)
    f0 = degb / (jnp.float32(_E / _N) + jnp.float32(1e-6))
    f1 = jnp.log(degb + 1.0)
    f2 = lax.rsqrt(jnp.maximum(degb, 1.0))
    f3 = (degb > med).astype(_f32)
    t = jnp.dot(xn, w0x_ref[:], preferred_element_type=_f32, precision=_HI)
    t = (t + f0 * w0f_ref[0:1, :] + f1 * w0f_ref[1:2, :]
         + f2 * w0f_ref[2:3, :] + f3 * w0f_ref[3:4, :] + b0_ref[:])
    t = jnp.maximum(_ln_tc(t, g0_ref[:], be0_ref[:]), 0.0)
    h0 = jnp.dot(t, w1_ref[:], preferred_element_type=_f32,
                 precision=_HI) + b1_ref[:]
    hw = jnp.dot(h0, wg_ref[:], preferred_element_type=_f32, precision=_HI)
    dinv = lax.rsqrt(cntb + 1.0)
    u = hw * dinv
    h0_ref[:] = h0
    u2_ref[0] = u[:, :128]
    u2_ref[1] = u[:, 128:]
    dinv_ref[:] = dinv


def _mid_body(y2_ref, u2_ref, h0_ref, dinv_ref, bg_ref,
              g1_ref, be1_ref, h12_ref):
    dinv = dinv_ref[:]
    agg = jnp.concatenate(
        [dinv * (y2_ref[0] + u2_ref[0]),
         dinv * (y2_ref[1] + u2_ref[1])], axis=1) + bg_ref[:]
    h1 = h0_ref[:] + jnp.maximum(_ln_tc(agg, g1_ref[:], be1_ref[:]), 0.0)
    h12_ref[0] = h1[:, :128]
    h12_ref[1] = h1[:, 128:]


def _post_body(s2_ref, cnt_ref, h0_ref, h12_ref,
               wl_ref, bl_ref, wr_ref, g2_ref, be2_ref,
               wjk0_ref, wjk1_ref, wjk2_ref, bjk_ref, z2_ref):
    inv_cnt = 1.0 / jnp.maximum(cnt_ref[:], 1.0)
    mean = jnp.concatenate([s2_ref[0], s2_ref[1]], axis=1) * inv_cnt
    h1 = jnp.concatenate([h12_ref[0], h12_ref[1]], axis=1)
    sage = (jnp.dot(mean, wl_ref[:], preferred_element_type=_f32,
                    precision=_HI) + bl_ref[:]
            + jnp.dot(h1, wr_ref[:], preferred_element_type=_f32,
                      precision=_HI))
    h2 = h1 + jnp.maximum(_ln_tc(sage, g2_ref[:], be2_ref[:]), 0.0)
    z = (jnp.dot(h0_ref[:], wjk0_ref[:], preferred_element_type=_f32,
                 precision=_HI)
         + jnp.dot(h1, wjk1_ref[:], preferred_element_type=_f32,
                   precision=_HI)
         + jnp.dot(h2, wjk2_ref[:], preferred_element_type=_f32,
                   precision=_HI)
         + bjk_ref[:])
    z2_ref[0] = z[:, :128]
    z2_ref[1] = z[:, 128:]


def _dec_body(ga_ref, gb_ref, wd0l_ref, wd0h_ref,
              bd0_ref, wd1_ref, bd1_ref, wd2_ref, bd2_ref, out_ref):
    hlo = ga_ref[0] * gb_ref[0]
    hhi = ga_ref[1] * gb_ref[1]
    t = (jnp.dot(hlo, wd0l_ref[:], preferred_element_type=_f32)
         + jnp.dot(hhi, wd0h_ref[:], preferred_element_type=_f32)
         + bd0_ref[:])
    t = jnp.maximum(t, 0.0)
    t = jnp.dot(t, wd1_ref[:], preferred_element_type=_f32) + bd1_ref[:]
    t = jnp.maximum(t, 0.0)
    out_ref[:] = (jnp.sum(t * wd2_ref[:], axis=1, keepdims=True)
                  + bd2_ref[0, 0])


def _full(shape):
    return pl.BlockSpec(shape, lambda *i: tuple(0 for _ in shape))


def _rows(shape):
    return pl.BlockSpec(shape, lambda i: (i,) + tuple(0 for _ in shape[1:]))


def _stk(shape):
    return pl.BlockSpec(shape, lambda i: (0, i, 0))


def kernel(x, edge_index, edge_pairs, W_in0, b_in0, g0, be0, W_in1, b_in1,
           W_gcn, b_gcn, g1, be1, W_l, b_l, W_r, g2, be2, W_jk, b_jk,
           W_d0, b_d0, W_d1, b_d1, W_d2, b_d2):
    pad_e = 10000 + (jnp.arange(_EP - _E, dtype=jnp.int32) % 240)
    edge4d = jnp.concatenate(
        [edge_index, jnp.stack([pad_e, pad_e])], axis=1
    ).reshape(2, 16, _ECH_T, 128)
    pad_p = 10000 + (jnp.arange(_PP - _P, dtype=jnp.int32) % 240)
    pair4d = jnp.stack(
        [jnp.concatenate([edge_pairs[:, 0], pad_p]),
         jnp.concatenate([edge_pairs[:, 1], pad_p])]
    ).reshape(2, 16, _PCH_T, 128)
    xp = jnp.pad(x, ((0, _NP - _N), (0, 0)))

    w0x = W_in0[:128]
    w0f = jnp.pad(W_in0[128:], ((0, 4), (0, 0)))
    row = lambda v: v.reshape(1, -1)

    hist = _sc_hist(edge4d)
    deg, cnt = hist[:_NP, 0], hist[_NP:, 0]
    deg1 = deg.reshape(_NP, 1)
    cnt1 = cnt.reshape(_NP, 1)

    med = pl.pallas_call(
        _median_body,
        out_shape=jax.ShapeDtypeStruct((1, 1), _f32),
        in_specs=[_full((80, 128))],
        out_specs=pl.BlockSpec(memory_space=pltpu.SMEM),
    )(deg.reshape(80, 128))

    grid = (_NP // 256,)
    h0, u2, dinv = pl.pallas_call(
        _prep_body,
        grid=grid,
        out_shape=[
            jax.ShapeDtypeStruct((_NP, 256), _f32),
            jax.ShapeDtypeStruct((2, _NP, 128), _f32),
            jax.ShapeDtypeStruct((_NP, 1), _f32),
        ],
        in_specs=[
            _rows((256, 128)), _rows((256, 1)), _rows((256, 1)),
            pl.BlockSpec(memory_space=pltpu.SMEM),
            _full((128, 256)), _full((8, 256)),
            _full((1, 256)), _full((1, 256)), _full((1, 256)),
            _full((256, 256)), _full((1, 256)), _full((256, 256)),
        ],
        out_specs=[
            _rows((256, 256)), _stk((2, 256, 128)), _rows((256, 1)),
        ],
    )(xp, deg1, cnt1, med, w0x, w0f, row(b_in0), row(g0), row(be0),
      W_in1, row(b_in1), W_gcn)

    y2n = _sc_scatter(u2.reshape(2 * _NP, 128), edge4d)

    h12 = pl.pallas_call(
        _mid_body,
        grid=grid,
        out_shape=jax.ShapeDtypeStruct((2, _NP, 128), _f32),
        in_specs=[
            _stk((2, 256, 128)), _stk((2, 256, 128)),
            _rows((256, 256)), _rows((256, 1)),
            _full((1, 256)), _full((1, 256)), _full((1, 256)),
        ],
        out_specs=_stk((2, 256, 128)),
    )(y2n.reshape(2, _NP, 128), u2, h0, dinv, row(b_gcn), row(g1), row(be1))

    s2n = _sc_scatter(h12.reshape(2 * _NP, 128), edge4d)

    z2 = pl.pallas_call(
        _post_body,
        grid=grid,
        out_shape=jax.ShapeDtypeStruct((2, _NP, 128), _f32),
        in_specs=[
            _stk((2, 256, 128)), _rows((256, 1)),
            _rows((256, 256)), _stk((2, 256, 128)),
            _full((256, 256)), _full((1, 256)), _full((256, 256)),
            _full((1, 256)), _full((1, 256)),
            _full((256, 256)), _full((256, 256)), _full((256, 256)),
            _full((1, 256)),
        ],
        out_specs=_stk((2, 256, 128)),
    )(s2n.reshape(2, _NP, 128), cnt1, h0, h12, W_l, row(b_l), W_r, row(g2),
      row(be2), W_jk[:256], W_jk[256:512], W_jk[512:], row(b_jk))

    ga, gb = _sc_pair_gather(z2.reshape(2 * _NP, 128), pair4d)

    out = pl.pallas_call(
        _dec_body,
        grid=(_PP // 512,),
        out_shape=jax.ShapeDtypeStruct((_PP, 1), _f32),
        in_specs=[
            _stk((2, 512, 128)), _stk((2, 512, 128)),
            _full((128, 256)), _full((128, 256)), _full((1, 256)),
            _full((256, 128)), _full((1, 128)), _full((1, 128)),
            _full((1, 1)),
        ],
        out_specs=_rows((512, 1)),
    )(ga, gb, W_d0[:128], W_d0[128:], row(b_d0),
      W_d1, row(b_d1), W_d2.reshape(1, 128), b_d2.reshape(1, 1))

    return out[:_P, 0]


# speedup vs baseline: 9.1368x; 1.0302x over previous
import functools

import jax
import jax.numpy as jnp
from jax import lax
from jax.experimental import pallas as pl
from jax.experimental.pallas import tpu as pltpu
from jax.experimental.pallas import tpu_sc as plsc

_N = 10000
_E = 320000
_P = 100000
_NP = 10240
_EP = 327680
_ECH_T = 160
_PP = 102400
_PPH = 51200
_PCH_T = 25

_mesh = plsc.VectorSubcoreMesh(core_axis_name="c", subcore_axis_name="s")
_f32 = jnp.float32


@functools.partial(
    pl.kernel,
    mesh=_mesh,
    out_type=jax.ShapeDtypeStruct((2 * _NP, 128), _f32),
    scratch_types=[
        pltpu.VMEM((_ECH_T, 128), jnp.int32),
        pltpu.VMEM((128, 128), _f32),
        pltpu.VMEM_SHARED((_NP, 128), _f32),
    ],
)
def _sc_hist(edge4d, hist_out, idx_v, buf, hist_spm):
    c = lax.axis_index("c")
    s = lax.axis_index("s")

    def _fill(val):
        def _fi(i, carry):
            for k in range(8):
                buf[i, pl.ds(k * 16, 16)] = jnp.full((16,), val, _f32)
            return carry

        lax.fori_loop(0, 128, _fi, 0)

    _fill(0.0)
    for k in range(5):
        pltpu.sync_copy(buf, hist_spm.at[pl.ds(s * 640 + k * 128, 128)])
    _fill(1.0)
    pltpu.sync_copy(edge4d.at[c, s], idx_v)
    plsc.subcore_barrier()

    def _body(j, carry):
        pltpu.sync_copy(buf, hist_spm.at[idx_v.at[j]], add=True)
        return carry

    lax.fori_loop(0, _ECH_T, _body, 0)
    plsc.subcore_barrier()
    pltpu.sync_copy(hist_spm.at[pl.ds(s * 640, 640)],
                    hist_out.at[pl.ds(c * _NP + s * 640, 640)])


@functools.partial(
    pl.kernel,
    mesh=_mesh,
    out_type=jax.ShapeDtypeStruct((2 * _NP, 128), _f32),
    scratch_types=[
        pltpu.VMEM((40, 128), jnp.int32),
        pltpu.VMEM((40, 128), jnp.int32),
        pltpu.VMEM((128, 128), _f32),
        pltpu.VMEM((128, 128), _f32),
        pltpu.VMEM_SHARED((_NP, 128), _f32),
        pltpu.SemaphoreType.DMA,
        pltpu.SemaphoreType.DMA,
    ],
)
def _sc_scatter(tab2n, edge4d, out2n, sidx, didx, rows0, rows1, acc,
                semA, semB):
    c = lax.axis_index("c")
    s = lax.axis_index("s")
    off = c * _NP

    def _zrow(i, carry):
        for k in range(8):
            rows0[i, pl.ds(k * 16, 16)] = jnp.zeros((16,), _f32)
        return carry

    lax.fori_loop(0, 128, _zrow, 0)
    for k in range(5):
        pltpu.sync_copy(rows0, acc.at[pl.ds(s * 640 + k * 128, 128)])
    plsc.subcore_barrier()

    def _adj(i, carry):
        for k in range(8):
            sidx[i, pl.ds(k * 16, 16)] = sidx[i, pl.ds(k * 16, 16)] + off
        return carry

    def _body(i, carry):
        j = 2 * i
        pltpu.make_async_copy(tab2n.at[sidx.at[j]], rows0, semA).wait()
        pltpu.async_copy(tab2n.at[sidx.at[j + 1]], rows1, semB)
        pltpu.sync_copy(rows0, acc.at[didx.at[j]], add=True)
        pltpu.make_async_copy(tab2n.at[sidx.at[j + 1]], rows1, semB).wait()

        @pl.when(j + 2 < 40)
        def _():
            pltpu.async_copy(tab2n.at[sidx.at[j + 2]], rows0, semA)

        pltpu.sync_copy(rows1, acc.at[didx.at[j + 1]], add=True)
        return carry

    for q in range(4):
        pltpu.sync_copy(edge4d.at[0, s, pl.ds(q * 40, 40)], sidx)
        pltpu.sync_copy(edge4d.at[1, s, pl.ds(q * 40, 40)], didx)
        lax.fori_loop(0, 40, _adj, 0)
        pltpu.async_copy(tab2n.at[sidx.at[0]], rows0, semA)
        lax.fori_loop(0, 20, _body, 0)

    plsc.subcore_barrier()
    pltpu.sync_copy(acc.at[pl.ds(s * 640, 640)],
                    out2n.at[pl.ds(c * _NP + s * 640, 640)])


@functools.partial(
    pl.kernel,
    mesh=_mesh,
    out_type=[
        jax.ShapeDtypeStruct((2, _PPH, 128), _f32),
        jax.ShapeDtypeStruct((2, _PPH, 128), _f32),
    ],
    scratch_types=[
        pltpu.VMEM((_PCH_T, 128), jnp.int32),
        pltpu.VMEM((_PCH_T, 128), jnp.int32),
        pltpu.VMEM((128, 128), _f32),
        pltpu.VMEM((128, 128), _f32),
        pltpu.SemaphoreType.DMA,
        pltpu.SemaphoreType.DMA,
    ],
)
def _sc_pair_gather(z2n, pair4d, out_a, out_b, aidx, bidx, rows0, rows1,
                    semA, semB):
    c = lax.axis_index("c")
    s = lax.axis_index("s")
    off = c * _NP
    pltpu.sync_copy(pair4d.at[0, s], aidx)
    pltpu.sync_copy(pair4d.at[1, s], bidx)

    def _adj(i, carry):
        for k in range(8):
            aidx[i, pl.ds(k * 16, 16)] = aidx[i, pl.ds(k * 16, 16)] + off
            bidx[i, pl.ds(k * 16, 16)] = bidx[i, pl.ds(k * 16, 16)] + off
        return carry

    lax.fori_loop(0, _PCH_T, _adj, 0)
    pltpu.async_copy(z2n.at[aidx.at[0]], rows0, semA)

    def _body(j, carry):
        base = (s * _PCH_T + j) * 128
        pltpu.make_async_copy(z2n.at[aidx.at[j]], rows0, semA).wait()
        pltpu.async_copy(z2n.at[bidx.at[j]], rows1, semB)
        pltpu.sync_copy(rows0, out_a.at[c, pl.ds(base, 128)])
        pltpu.make_async_copy(z2n.at[bidx.at[j]], rows1, semB).wait()

        @pl.when(j + 1 < _PCH_T)
        def _():
            pltpu.async_copy(z2n.at[aidx.at[j + 1]], rows0, semA)

        pltpu.sync_copy(rows1, out_b.at[c, pl.ds(base, 128)])
        return carry

    lax.fori_loop(0, _PCH_T, _body, 0)


_HI = lax.Precision.HIGHEST


def _ln_tc(h, g, b):
    mu = jnp.sum(h, axis=1, keepdims=True) * (1.0 / 256.0)
    d = h - mu
    var = jnp.sum(d * d, axis=1, keepdims=True) * (1.0 / 256.0)
    return d * lax.rsqrt(var + 1e-5) * g + b


def _median_body(deg_ref, med_ref):
    deg = deg_ref[:]
    flat = (lax.broadcasted_iota(jnp.int32, (80, 128), 0) * 128
            + lax.broadcasted_iota(jnp.int32, (80, 128), 1))
    valid = flat < _N

    def _body(i, lohi):
        lo, hi = lohi
        mid = (lo + hi) // 2
        cnt = jnp.sum(jnp.where(valid & (deg <= mid.astype(_f32)), 1, 0))
        ge = cnt >= (_N - 1) // 2 + 1
        return (jnp.where(ge, lo, mid + 1), jnp.where(ge, mid, hi))

    lo, _hi = lax.fori_loop(0, 19, _body,
                            (jnp.int32(0), jnp.int32(_E)))
    med_ref[0, 0] = lo.astype(_f32)


def _prep_body(x_ref, deg_ref, cnt_ref, med_ref, w0x_ref, w0f_ref, b0_ref,
               g0_ref, be0_ref, w1_ref, b1_ref, wg_ref,
               h0_ref, u2_ref, dinv_ref):
    xb = x_ref[:]
    degb = deg_ref[:]
    cntb = cnt_ref[:]
    med = med_ref[0, 0]
    nrm = jnp.sqrt(jnp.sum(xb * xb, axis=1, keepdims=True))
    xn = xb / jnp.maximum(nrm, 1e-12)
    f0 = degb / (jnp.float32(_E / _N) + jnp.float32(1e-6))
    f1 = jnp.log(degb + 1.0)
    f2 = lax.rsqrt(jnp.maximum(degb, 1.0))
    f3 = (degb > med).astype(_f32)
    t = jnp.dot(xn, w0x_ref[:], preferred_element_type=_f32, precision=_HI)
    t = (t + f0 * w0f_ref[0:1, :] + f1 * w0f_ref[1:2, :]
         + f2 * w0f_ref[2:3, :] + f3 * w0f_ref[3:4, :] + b0_ref[:])
    t = jnp.maximum(_ln_tc(t, g0_ref[:], be0_ref[:]), 0.0)
    h0 = jnp.dot(t, w1_ref[:], preferred_element_type=_f32,
                 precision=_HI) + b1_ref[:]
    hw = jnp.dot(h0, wg_ref[:], preferred_element_type=_f32, precision=_HI)
    dinv = lax.rsqrt(cntb + 1.0)
    u = hw * dinv
    h0_ref[:] = h0
    u2_ref[0] = u[:, :128]
    u2_ref[1] = u[:, 128:]
    dinv_ref[:] = dinv


def _mid_body(y2_ref, u2_ref, h0_ref, dinv_ref, bg_ref,
              g1_ref, be1_ref, h12_ref):
    dinv = dinv_ref[:]
    agg = jnp.concatenate(
        [dinv * (y2_ref[0] + u2_ref[0]),
         dinv * (y2_ref[1] + u2_ref[1])], axis=1) + bg_ref[:]
    h1 = h0_ref[:] + jnp.maximum(_ln_tc(agg, g1_ref[:], be1_ref[:]), 0.0)
    h12_ref[0] = h1[:, :128]
    h12_ref[1] = h1[:, 128:]


def _post_body(s2_ref, cnt_ref, h0_ref, h12_ref,
               wl_ref, bl_ref, wr_ref, g2_ref, be2_ref,
               wjk0_ref, wjk1_ref, wjk2_ref, bjk_ref, z2_ref):
    inv_cnt = 1.0 / jnp.maximum(cnt_ref[:], 1.0)
    mean = jnp.concatenate([s2_ref[0], s2_ref[1]], axis=1) * inv_cnt
    h1 = jnp.concatenate([h12_ref[0], h12_ref[1]], axis=1)
    sage = (jnp.dot(mean, wl_ref[:], preferred_element_type=_f32,
                    precision=_HI) + bl_ref[:]
            + jnp.dot(h1, wr_ref[:], preferred_element_type=_f32,
                      precision=_HI))
    h2 = h1 + jnp.maximum(_ln_tc(sage, g2_ref[:], be2_ref[:]), 0.0)
    z = (jnp.dot(h0_ref[:], wjk0_ref[:], preferred_element_type=_f32,
                 precision=_HI)
         + jnp.dot(h1, wjk1_ref[:], preferred_element_type=_f32,
                   precision=_HI)
         + jnp.dot(h2, wjk2_ref[:], preferred_element_type=_f32,
                   precision=_HI)
         + bjk_ref[:])
    z2_ref[0] = z[:, :128]
    z2_ref[1] = z[:, 128:]


def _dec_body(ga_ref, gb_ref, wd0l_ref, wd0h_ref,
              bd0_ref, wd1_ref, bd1_ref, wd2_ref, bd2_ref, out_ref):
    hlo = ga_ref[0] * gb_ref[0]
    hhi = ga_ref[1] * gb_ref[1]
    t = (jnp.dot(hlo, wd0l_ref[:], preferred_element_type=_f32)
         + jnp.dot(hhi, wd0h_ref[:], preferred_element_type=_f32)
         + bd0_ref[:])
    t = jnp.maximum(t, 0.0)
    t = jnp.dot(t, wd1_ref[:], preferred_element_type=_f32) + bd1_ref[:]
    t = jnp.maximum(t, 0.0)
    out_ref[:] = (jnp.sum(t * wd2_ref[:], axis=1, keepdims=True)
                  + bd2_ref[0, 0])


def _full(shape):
    return pl.BlockSpec(shape, lambda *i: tuple(0 for _ in shape))


def _rows(shape):
    return pl.BlockSpec(shape, lambda i: (i,) + tuple(0 for _ in shape[1:]))


def _stk(shape):
    return pl.BlockSpec(shape, lambda i: (0, i, 0))


def kernel(x, edge_index, edge_pairs, W_in0, b_in0, g0, be0, W_in1, b_in1,
           W_gcn, b_gcn, g1, be1, W_l, b_l, W_r, g2, be2, W_jk, b_jk,
           W_d0, b_d0, W_d1, b_d1, W_d2, b_d2):
    pad_e = 10000 + (jnp.arange(_EP - _E, dtype=jnp.int32) % 240)
    edge4d = jnp.concatenate(
        [edge_index, jnp.stack([pad_e, pad_e])], axis=1
    ).reshape(2, 16, _ECH_T, 128)
    pad_p = 10000 + (jnp.arange(_PP - _P, dtype=jnp.int32) % 240)
    pa = jnp.concatenate([edge_pairs[:, 0], pad_p])
    pb = jnp.concatenate([edge_pairs[:, 1], pad_p])
    pair_h1 = jnp.stack([pa[:_PPH], pb[:_PPH]]).reshape(2, 16, _PCH_T, 128)
    pair_h2 = jnp.stack([pa[_PPH:], pb[_PPH:]]).reshape(2, 16, _PCH_T, 128)
    xp = jnp.pad(x, ((0, _NP - _N), (0, 0)))

    w0x = W_in0[:128]
    w0f = jnp.pad(W_in0[128:], ((0, 4), (0, 0)))
    row = lambda v: v.reshape(1, -1)

    hist = _sc_hist(edge4d)
    deg, cnt = hist[:_NP, 0], hist[_NP:, 0]
    deg1 = deg.reshape(_NP, 1)
    cnt1 = cnt.reshape(_NP, 1)

    med = pl.pallas_call(
        _median_body,
        out_shape=jax.ShapeDtypeStruct((1, 1), _f32),
        in_specs=[_full((80, 128))],
        out_specs=pl.BlockSpec(memory_space=pltpu.SMEM),
    )(deg.reshape(80, 128))

    grid = (_NP // 256,)
    h0, u2, dinv = pl.pallas_call(
        _prep_body,
        grid=grid,
        out_shape=[
            jax.ShapeDtypeStruct((_NP, 256), _f32),
            jax.ShapeDtypeStruct((2, _NP, 128), _f32),
            jax.ShapeDtypeStruct((_NP, 1), _f32),
        ],
        in_specs=[
            _rows((256, 128)), _rows((256, 1)), _rows((256, 1)),
            pl.BlockSpec(memory_space=pltpu.SMEM),
            _full((128, 256)), _full((8, 256)),
            _full((1, 256)), _full((1, 256)), _full((1, 256)),
            _full((256, 256)), _full((1, 256)), _full((256, 256)),
        ],
        out_specs=[
            _rows((256, 256)), _stk((2, 256, 128)), _rows((256, 1)),
        ],
    )(xp, deg1, cnt1, med, w0x, w0f, row(b_in0), row(g0), row(be0),
      W_in1, row(b_in1), W_gcn)

    y2n = _sc_scatter(u2.reshape(2 * _NP, 128), edge4d)

    h12 = pl.pallas_call(
        _mid_body,
        grid=grid,
        out_shape=jax.ShapeDtypeStruct((2, _NP, 128), _f32),
        in_specs=[
            _stk((2, 256, 128)), _stk((2, 256, 128)),
            _rows((256, 256)), _rows((256, 1)),
            _full((1, 256)), _full((1, 256)), _full((1, 256)),
        ],
        out_specs=_stk((2, 256, 128)),
    )(y2n.reshape(2, _NP, 128), u2, h0, dinv, row(b_gcn), row(g1), row(be1))

    s2n = _sc_scatter(h12.reshape(2 * _NP, 128), edge4d)

    z2 = pl.pallas_call(
        _post_body,
        grid=grid,
        out_shape=jax.ShapeDtypeStruct((2, _NP, 128), _f32),
        in_specs=[
            _stk((2, 256, 128)), _rows((256, 1)),
            _rows((256, 256)), _stk((2, 256, 128)),
            _full((256, 256)), _full((1, 256)), _full((256, 256)),
            _full((1, 256)), _full((1, 256)),
            _full((256, 256)), _full((256, 256)), _full((256, 256)),
            _full((1, 256)),
        ],
        out_specs=_stk((2, 256, 128)),
    )(s2n.reshape(2, _NP, 128), cnt1, h0, h12, W_l, row(b_l), W_r, row(g2),
      row(be2), W_jk[:256], W_jk[256:512], W_jk[512:], row(b_jk))

    z2n = z2.reshape(2 * _NP, 128)

    def _decode(ga, gb):
        return pl.pallas_call(
            _dec_body,
            grid=(_PPH // 512,),
            out_shape=jax.ShapeDtypeStruct((_PPH, 1), _f32),
            in_specs=[
                _stk((2, 512, 128)), _stk((2, 512, 128)),
                _full((128, 256)), _full((128, 256)), _full((1, 256)),
                _full((256, 128)), _full((1, 128)), _full((1, 128)),
                _full((1, 1)),
            ],
            out_specs=_rows((512, 1)),
        )(ga, gb, W_d0[:128], W_d0[128:], row(b_d0),
          W_d1, row(b_d1), W_d2.reshape(1, 128), b_d2.reshape(1, 1))

    ga1, gb1 = _sc_pair_gather(z2n, pair_h1)
    ga2, gb2 = _sc_pair_gather(z2n, pair_h2)
    out1 = _decode(ga1, gb1)
    out2 = _decode(ga2, gb2)

    return jnp.concatenate([out1, out2], axis=0)[:_P, 0]
